# baseline (speedup 1.0000x reference)
import functools

import jax
import jax.numpy as jnp
from jax.experimental import pallas as pl
from jax.experimental.pallas import tpu as pltpu


def _conv3x3_kernel(x_ref, w_ref, b_ref, o_ref, r1, r2, col, *,
                    N, c_in, bt, Lr):
    for bl in range(bt):
        r1[bl, :, :Lr - 1] = x_ref[bl, :, 1:Lr]
        r2[bl, :, :Lr - 2] = x_ref[bl, :, 2:Lr]
        srcs = (x_ref, r1, r2)
        for kh in range(3):
            for kw in range(3):
                t = kh * 3 + kw
                col[bl, t * c_in:(t + 1) * c_in, :] = srcs[kw][bl, :, kh * 64:kh * 64 + N]
        o_ref[bl] = (jnp.dot(w_ref[...], col[bl],
                             preferred_element_type=jnp.float32)
                     + b_ref[...]).astype(jnp.bfloat16)


def kernel(x, weight, bias):
    B, C_in, H, W = x.shape
    C_out, _, KH, KW = weight.shape
    assert (KH, KW) == (3, 3)
    Wp = 64
    N = H * Wp
    Hs = H + 3
    Lr = Hs * Wp

    bt = 1
    for d in (4, 2):
        if B % d == 0:
            bt = d
            break
    grid = (B // bt,)

    xs = jnp.pad(x, ((0, 0), (0, 0), (1, 2), (1, Wp - 1 - W)))
    xs = xs.astype(jnp.bfloat16).reshape(B, C_in, Lr)

    w2 = jnp.transpose(weight, (0, 2, 3, 1)).reshape(C_out, KH * KW * C_in)
    w2 = w2.astype(jnp.bfloat16)
    b2 = bias.astype(jnp.float32).reshape(C_out, 1)

    body = functools.partial(_conv3x3_kernel, N=N, c_in=C_in, bt=bt, Lr=Lr)

    out = pl.pallas_call(
        body,
        out_shape=jax.ShapeDtypeStruct((B, C_out, N), jnp.bfloat16),
        grid=grid,
        in_specs=[
            pl.BlockSpec((bt, C_in, Lr), lambda g: (g, 0, 0)),
            pl.BlockSpec((C_out, KH * KW * C_in), lambda g: (0, 0)),
            pl.BlockSpec((C_out, 1), lambda g: (0, 0)),
        ],
        out_specs=pl.BlockSpec((bt, C_out, N), lambda g: (g, 0, 0)),
        scratch_shapes=[
            pltpu.VMEM((bt, C_in, Lr), jnp.bfloat16),
            pltpu.VMEM((bt, C_in, Lr), jnp.bfloat16),
            pltpu.VMEM((bt, KH * KW * C_in, N), jnp.bfloat16),
        ],
        compiler_params=pltpu.CompilerParams(
            dimension_semantics=("parallel",)),
    )(xs, w2, b2)
    return out.reshape(B, C_out, H, Wp)[:, :, :, :W].astype(jnp.float32)


# speedup vs baseline: 1.1190x; 1.1190x over previous
import functools

import jax
import jax.numpy as jnp
from jax.experimental import pallas as pl
from jax.experimental.pallas import tpu as pltpu


def _conv3x3_kernel(x_ref, w_ref, b_ref, o_ref, slab, r1, r2, col, *,
                    H, W, N, c_in, bt, Lr):
    Wp = 64
    zp = jnp.zeros((c_in, Wp + 1), jnp.bfloat16)
    zg = jnp.zeros((c_in, Wp - W), jnp.bfloat16)
    zt = jnp.zeros((c_in, Lr - (H + 1) * Wp - 1), jnp.bfloat16)
    for bl in range(bt):
        slab[bl, :, :Wp + 1] = zp
        for r in range(1, H + 1):
            slab[bl, :, r * Wp + 1 + W:r * Wp + Wp + 1] = zg
        slab[bl, :, (H + 1) * Wp + 1:] = zt
        for h in range(H):
            base = (h + 1) * Wp + 1
            slab[bl, :, base:base + W] = x_ref[bl, :, h * W:(h + 1) * W].astype(jnp.bfloat16)
        r1[bl, :, :Lr - 1] = slab[bl, :, 1:Lr]
        r2[bl, :, :Lr - 2] = slab[bl, :, 2:Lr]
        srcs = (slab, r1, r2)
        for kh in range(3):
            for kw in range(3):
                t = kh * 3 + kw
                col[bl, t * c_in:(t + 1) * c_in, :] = srcs[kw][bl, :, kh * 64:kh * 64 + N]
        o_ref[bl] = (jnp.dot(w_ref[...], col[bl],
                             preferred_element_type=jnp.float32)
                     + b_ref[...]).astype(jnp.bfloat16)


def kernel(x, weight, bias):
    B, C_in, H, W = x.shape
    C_out, _, KH, KW = weight.shape
    assert (KH, KW) == (3, 3)
    Wp = 64
    N = H * Wp
    Lr = (H + 3) * Wp

    bt = 1
    for d in (4, 2):
        if B % d == 0:
            bt = d
            break
    grid = (B // bt,)

    x2 = x.reshape(B, C_in, H * W)
    w2 = jnp.transpose(weight, (0, 2, 3, 1)).reshape(C_out, KH * KW * C_in)
    w2 = w2.astype(jnp.bfloat16)
    b2 = bias.astype(jnp.float32).reshape(C_out, 1)

    body = functools.partial(_conv3x3_kernel, H=H, W=W, N=N,
                             c_in=C_in, bt=bt, Lr=Lr)

    out = pl.pallas_call(
        body,
        out_shape=jax.ShapeDtypeStruct((B, C_out, N), jnp.bfloat16),
        grid=grid,
        in_specs=[
            pl.BlockSpec((bt, C_in, H * W), lambda g: (g, 0, 0)),
            pl.BlockSpec((C_out, KH * KW * C_in), lambda g: (0, 0)),
            pl.BlockSpec((C_out, 1), lambda g: (0, 0)),
        ],
        out_specs=pl.BlockSpec((bt, C_out, N), lambda g: (g, 0, 0)),
        scratch_shapes=[
            pltpu.VMEM((bt, C_in, Lr), jnp.bfloat16),
            pltpu.VMEM((bt, C_in, Lr), jnp.bfloat16),
            pltpu.VMEM((bt, C_in, Lr), jnp.bfloat16),
            pltpu.VMEM((bt, KH * KW * C_in, N), jnp.bfloat16),
        ],
        compiler_params=pltpu.CompilerParams(
            dimension_semantics=("parallel",)),
    )(x2, w2, b2)
    return out.reshape(B, C_out, H, Wp)[:, :, :, :W].astype(jnp.float32)


# speedup vs baseline: 1.1779x; 1.0526x over previous
import functools

import jax
import jax.numpy as jnp
from jax.experimental import pallas as pl
from jax.experimental.pallas import tpu as pltpu


def _conv3x3_kernel(x_ref, w_ref, b_ref, o_ref, slab, r1, r2, col, accs, *,
                    H, W, N, c_in, bt, Lr):
    Wp = 64
    zp = jnp.zeros((c_in, Wp + 1), jnp.bfloat16)
    zg = jnp.zeros((c_in, Wp - W), jnp.bfloat16)
    zt = jnp.zeros((c_in, Lr - (H + 1) * Wp - 1), jnp.bfloat16)
    for bl in range(bt):
        slab[bl, :, :Wp + 1] = zp
        for r in range(1, H + 1):
            slab[bl, :, r * Wp + 1 + W:r * Wp + Wp + 1] = zg
        slab[bl, :, (H + 1) * Wp + 1:] = zt
        for h in range(H):
            base = (h + 1) * Wp + 1
            slab[bl, :, base:base + W] = x_ref[bl, :, h * W:(h + 1) * W].astype(jnp.bfloat16)
        r1[bl, :, :Lr - 1] = slab[bl, :, 1:Lr]
        r2[bl, :, :Lr - 2] = slab[bl, :, 2:Lr]
        srcs = (slab, r1, r2)
        for kh in range(3):
            for kw in range(3):
                t = kh * 3 + kw
                col[bl, t * c_in:(t + 1) * c_in, :] = srcs[kw][bl, :, kh * 64:kh * 64 + N]
        accs[bl] = jnp.dot(w_ref[...], col[bl],
                           preferred_element_type=jnp.float32) + b_ref[...]
    for bl in range(bt):
        for h in range(H):
            o_ref[bl, :, h * W:(h + 1) * W] = accs[bl, :, h * 64:h * 64 + W]


def kernel(x, weight, bias):
    B, C_in, H, W = x.shape
    C_out, _, KH, KW = weight.shape
    assert (KH, KW) == (3, 3)
    Wp = 64
    N = H * Wp
    Lr = (H + 3) * Wp

    bt = 1
    for d in (4, 2):
        if B % d == 0:
            bt = d
            break
    grid = (B // bt,)

    x2 = x.reshape(B, C_in, H * W)
    w2 = jnp.transpose(weight, (0, 2, 3, 1)).reshape(C_out, KH * KW * C_in)
    w2 = w2.astype(jnp.bfloat16)
    b2 = bias.astype(jnp.float32).reshape(C_out, 1)

    body = functools.partial(_conv3x3_kernel, H=H, W=W, N=N,
                             c_in=C_in, bt=bt, Lr=Lr)

    out = pl.pallas_call(
        body,
        out_shape=jax.ShapeDtypeStruct((B, C_out, H * W), jnp.float32),
        grid=grid,
        in_specs=[
            pl.BlockSpec((bt, C_in, H * W), lambda g: (g, 0, 0)),
            pl.BlockSpec((C_out, KH * KW * C_in), lambda g: (0, 0)),
            pl.BlockSpec((C_out, 1), lambda g: (0, 0)),
        ],
        out_specs=pl.BlockSpec((bt, C_out, H * W), lambda g: (g, 0, 0)),
        scratch_shapes=[
            pltpu.VMEM((bt, C_in, Lr), jnp.bfloat16),
            pltpu.VMEM((bt, C_in, Lr), jnp.bfloat16),
            pltpu.VMEM((bt, C_in, Lr), jnp.bfloat16),
            pltpu.VMEM((bt, KH * KW * C_in, N), jnp.bfloat16),
            pltpu.VMEM((bt, C_out, N), jnp.float32),
        ],
        compiler_params=pltpu.CompilerParams(
            dimension_semantics=("parallel",)),
    )(x2, w2, b2)
    return out.reshape(B, C_out, H, W)
